# Initial kernel scaffold; baseline (speedup 1.0000x reference)
import jax
import jax.numpy as jnp
from jax.experimental import pallas as pl


def kernel(cycle_curve_data, logits, moe_masks, W, b):
    raise NotImplementedError("write your pallas kernel here")


# speedup vs baseline: 1.2962x; 1.2962x over previous
import functools

import jax
import jax.numpy as jnp
from jax.experimental import pallas as pl
from jax.experimental.pallas import tpu as pltpu

B, L, E, DIN, DM = 1024, 16, 8, 900, 768
TOP_K = 2
EPS = 1e-9


def _routing_body(logits_ref, masks_ref, gates_ref):
    logits = logits_ref[...]
    mask = jnp.where(masks_ref[...] == 1.0, 1.0, 0.0)
    m = jnp.max(logits, axis=1, keepdims=True)
    ex = jnp.exp(logits - m)
    probs = ex / jnp.sum(ex, axis=1, keepdims=True)
    g = probs * mask
    lane = jax.lax.broadcasted_iota(jnp.int32, (B, E), 1)
    m1 = jnp.max(g, axis=1, keepdims=True)
    i1 = jnp.min(jnp.where(g == m1, lane, E), axis=1, keepdims=True)
    g_wo = jnp.where(lane == i1, -jnp.inf, g)
    m2 = jnp.max(g_wo, axis=1, keepdims=True)
    i2 = jnp.min(jnp.where(g_wo == m2, lane, E), axis=1, keepdims=True)
    denom = m1 + m2 + EPS
    sel = (lane == i1) | (lane == i2)
    gates = jnp.where(sel, g / denom, 0.0)
    gexp = jnp.broadcast_to(gates[:, None, :], (B, L, E)).reshape(B * L, E)
    gates_ref[...] = gexp


def _moe_body(x_ref, w_ref, b_ref, gates_ref, out_ref, acc_ref):
    x = x_ref[...]
    acc_ref[...] = jnp.zeros_like(acc_ref)
    for e in range(E):
        y = jnp.dot(x, w_ref[e], preferred_element_type=jnp.float32)
        y = y + b_ref[e][None, :]
        ge = gates_ref[:, e][:, None]
        acc_ref[...] += ge * y
    out_ref[...] = acc_ref[...].astype(jnp.bfloat16)


def kernel(cycle_curve_data, logits, moe_masks, W, b):
    gates = pl.pallas_call(
        _routing_body,
        out_shape=jax.ShapeDtypeStruct((B * L, E), jnp.float32),
    )(logits, moe_masks)

    xr = cycle_curve_data.reshape(B * L, DIN)
    rows_per_step = 512
    nsteps = (B * L) // rows_per_step
    out = pl.pallas_call(
        _moe_body,
        grid=(nsteps,),
        in_specs=[
            pl.BlockSpec((rows_per_step, DIN), lambda i: (i, 0)),
            pl.BlockSpec((E, DIN, DM), lambda i: (0, 0, 0)),
            pl.BlockSpec((E, DM), lambda i: (0, 0)),
            pl.BlockSpec((rows_per_step, E), lambda i: (i, 0)),
        ],
        out_specs=pl.BlockSpec((rows_per_step, DM), lambda i: (i, 0)),
        out_shape=jax.ShapeDtypeStruct((B * L, DM), jnp.bfloat16),
        scratch_shapes=[pltpu.VMEM((rows_per_step, DM), jnp.float32)],
    )(xr, W, b, gates)
    return out.reshape(B, L, DM)


# speedup vs baseline: 1.3162x; 1.0154x over previous
import functools

import jax
import jax.numpy as jnp
from jax.experimental import pallas as pl
from jax.experimental.pallas import tpu as pltpu

B, L, E, DIN, DM = 1024, 16, 8, 900, 768
TOP_K = 2
EPS = 1e-9


def _routing_body(logits_ref, masks_ref, gates_ref):
    logits = logits_ref[...]
    mask = jnp.where(masks_ref[...] == 1.0, 1.0, 0.0)
    m = jnp.max(logits, axis=1, keepdims=True)
    ex = jnp.exp(logits - m)
    probs = ex / jnp.sum(ex, axis=1, keepdims=True)
    g = probs * mask
    lane = jax.lax.broadcasted_iota(jnp.int32, (B, E), 1)
    m1 = jnp.max(g, axis=1, keepdims=True)
    i1 = jnp.min(jnp.where(g == m1, lane, E), axis=1, keepdims=True)
    g_wo = jnp.where(lane == i1, -jnp.inf, g)
    m2 = jnp.max(g_wo, axis=1, keepdims=True)
    i2 = jnp.min(jnp.where(g_wo == m2, lane, E), axis=1, keepdims=True)
    denom = m1 + m2 + EPS
    sel = (lane == i1) | (lane == i2)
    gates = jnp.where(sel, g / denom, 0.0)
    gexp = jnp.broadcast_to(gates[:, None, :], (B, L, E)).reshape(B * L, E)
    gates_ref[...] = gexp


def _moe_body(x_ref, w_ref, b_ref, gates_ref, out_ref, acc_ref):
    x = x_ref[...]
    acc_ref[...] = jnp.zeros_like(acc_ref)
    for e in range(E):
        y = jnp.dot(x, w_ref[e], preferred_element_type=jnp.float32)
        y = y + b_ref[e][None, :]
        ge = gates_ref[:, e][:, None]
        acc_ref[...] += ge * y
    out_ref[...] = acc_ref[...].astype(jnp.bfloat16)


def kernel(cycle_curve_data, logits, moe_masks, W, b):
    gates = pl.pallas_call(
        _routing_body,
        out_shape=jax.ShapeDtypeStruct((B * L, E), jnp.float32),
    )(logits, moe_masks)

    xr = cycle_curve_data.reshape(B * L, DIN).astype(jnp.bfloat16)
    Wb = W.astype(jnp.bfloat16)
    rows_per_step = 512
    nsteps = (B * L) // rows_per_step
    out = pl.pallas_call(
        _moe_body,
        grid=(nsteps,),
        in_specs=[
            pl.BlockSpec((rows_per_step, DIN), lambda i: (i, 0)),
            pl.BlockSpec((E, DIN, DM), lambda i: (0, 0, 0)),
            pl.BlockSpec((E, DM), lambda i: (0, 0)),
            pl.BlockSpec((rows_per_step, E), lambda i: (i, 0)),
        ],
        out_specs=pl.BlockSpec((rows_per_step, DM), lambda i: (i, 0)),
        out_shape=jax.ShapeDtypeStruct((B * L, DM), jnp.bfloat16),
        scratch_shapes=[pltpu.VMEM((rows_per_step, DM), jnp.float32)],
    )(xr, Wb, b, gates)
    return out.reshape(B, L, DM)
